# Initial kernel scaffold; baseline (speedup 1.0000x reference)
import jax
import jax.numpy as jnp
from jax.experimental import pallas as pl


def kernel(coord, embeddings_table, special_token):
    raise NotImplementedError("write your pallas kernel here")


# speedup vs baseline: 6.1794x; 6.1794x over previous
import functools

import jax
import jax.numpy as jnp
from jax import lax
from jax.experimental import pallas as pl
from jax.experimental.pallas import tpu as pltpu
from jax.experimental.pallas import tpu_sc as plsc

N_POS_EMB = 100000
LANES = 16

NC = 2
NS = 16
NW = NC * NS

BATCH = 16384
HIST = 200
D = 16
B_TOTAL = BATCH * HIST
B_PER_W = B_TOTAL // NW
CHUNK = 2048
NCHUNK = B_PER_W // CHUNK


def _body(coord_hbm, table_hbm, out_hbm, coord_v, idx_v, rows_v, gsem):
    wid = lax.axis_index("s") * NC + lax.axis_index("c")
    base = wid * B_PER_W

    @pl.loop(0, NCHUNK)
    def _chunk(g):
        off = base + g * CHUNK
        pltpu.sync_copy(coord_hbm.at[pl.ds(off, CHUNK)], coord_v)

        @pl.loop(0, CHUNK // LANES)
        def _cvt(i):
            v = coord_v[pl.ds(i * LANES, LANES)]
            scaled = v * float(N_POS_EMB)
            clipped = jnp.minimum(jnp.maximum(scaled, 0.0), float(N_POS_EMB))
            idx_v[pl.ds(i * LANES, LANES)] = clipped.astype(jnp.int32)

        pltpu.async_copy(table_hbm.at[idx_v], rows_v, gsem).wait()
        pltpu.sync_copy(rows_v, out_hbm.at[pl.ds(off, CHUNK)])


def kernel(coord, embeddings_table, special_token):
    del special_token
    coord_flat = coord.reshape(B_TOTAL)

    mesh = plsc.VectorSubcoreMesh(core_axis_name="c", subcore_axis_name="s")
    run = pl.kernel(
        _body,
        out_type=jax.ShapeDtypeStruct((B_TOTAL, D), jnp.float32),
        mesh=mesh,
        scratch_types=[
            pltpu.VMEM((CHUNK,), jnp.float32),
            pltpu.VMEM((CHUNK,), jnp.int32),
            pltpu.VMEM((CHUNK, D), jnp.float32),
            pltpu.SemaphoreType.DMA,
        ],
        compiler_params=pltpu.CompilerParams(use_tc_tiling_on_sc=False),
    )
    out = run(coord_flat, embeddings_table)
    return out.reshape(BATCH, HIST, D)


# speedup vs baseline: 6.5388x; 1.0582x over previous
import functools

import jax
import jax.numpy as jnp
from jax import lax
from jax.experimental import pallas as pl
from jax.experimental.pallas import tpu as pltpu
from jax.experimental.pallas import tpu_sc as plsc

N_POS_EMB = 100000
LANES = 16

NC = 2
NS = 16
NW = NC * NS

BATCH = 16384
HIST = 200
D = 16
B_TOTAL = BATCH * HIST
B_PER_W = B_TOTAL // NW
CHUNK = 2048
NCHUNK = B_PER_W // CHUNK


def _body(coord_hbm, table_hbm, out_hbm, coord_v, idx_v, rows_v,
          csem0, csem1, gsem0, gsem1, osem0, osem1):
    csem = (csem0, csem1)
    gsem = (gsem0, gsem1)
    osem = (osem0, osem1)

    wid = lax.axis_index("s") * NC + lax.axis_index("c")
    base = wid * B_PER_W

    def coord_in(g, b):
        pltpu.async_copy(
            coord_hbm.at[pl.ds(base + g * CHUNK, CHUNK)], coord_v.at[b], csem[b]
        )

    def wait_coord(g, b):
        pltpu.make_async_copy(
            coord_hbm.at[pl.ds(base + g * CHUNK, CHUNK)], coord_v.at[b], csem[b]
        ).wait()

    def compute_idx(b):
        @pl.loop(0, CHUNK // LANES, unroll=4)
        def _cvt(i):
            v = coord_v[b, pl.ds(i * LANES, LANES)]
            scaled = v * float(N_POS_EMB)
            clipped = jnp.minimum(jnp.maximum(scaled, 0.0), float(N_POS_EMB))
            idx_v[b, pl.ds(i * LANES, LANES)] = clipped.astype(jnp.int32)

    def fire_gather(b):
        pltpu.async_copy(table_hbm.at[idx_v.at[b]], rows_v.at[b], gsem[b])

    def wait_gather(b):
        pltpu.make_async_copy(table_hbm.at[idx_v.at[b]], rows_v.at[b], gsem[b]).wait()

    def fire_out(g, b):
        pltpu.async_copy(
            rows_v.at[b], out_hbm.at[pl.ds(base + g * CHUNK, CHUNK)], osem[b]
        )

    def wait_out(g, b):
        pltpu.make_async_copy(
            rows_v.at[b], out_hbm.at[pl.ds(base + g * CHUNK, CHUNK)], osem[b]
        ).wait()

    coord_in(0, 0)

    @pl.loop(0, NCHUNK, step=2)
    def _chunk(g0):
        for b in range(2):
            g = g0 + b
            @pl.when(g0 >= 2 if b == 0 else g0 >= 1)
            def _():
                wait_out(g - 2, b)

            @pl.when(g + 1 < NCHUNK)
            def _():
                coord_in(g + 1, 1 - b)

            wait_coord(g, b)
            compute_idx(b)
            fire_gather(b)
            @pl.when(g >= 1)
            def _():
                wait_gather(1 - b)
                fire_out(g - 1, 1 - b)

    last = NCHUNK - 1
    wait_gather(last % 2)
    fire_out(last, last % 2)
    wait_out(last - 1, (last - 1) % 2)
    wait_out(last, last % 2)


def kernel(coord, embeddings_table, special_token):
    del special_token
    coord_flat = coord.reshape(B_TOTAL)

    mesh = plsc.VectorSubcoreMesh(core_axis_name="c", subcore_axis_name="s")
    run = pl.kernel(
        _body,
        out_type=jax.ShapeDtypeStruct((B_TOTAL, D), jnp.float32),
        mesh=mesh,
        scratch_types=[
            pltpu.VMEM((2, CHUNK), jnp.float32),
            pltpu.VMEM((2, CHUNK), jnp.int32),
            pltpu.VMEM((2, CHUNK, D), jnp.float32),
            pltpu.SemaphoreType.DMA,
            pltpu.SemaphoreType.DMA,
            pltpu.SemaphoreType.DMA,
            pltpu.SemaphoreType.DMA,
            pltpu.SemaphoreType.DMA,
            pltpu.SemaphoreType.DMA,
        ],
        compiler_params=pltpu.CompilerParams(use_tc_tiling_on_sc=False),
    )
    out = run(coord_flat, embeddings_table)
    return out.reshape(BATCH, HIST, D)
